# baseline (speedup 1.0000x reference)
import functools

import jax
import jax.numpy as jnp
from jax import lax
from jax.experimental import pallas as pl
from jax.experimental.pallas import tpu as pltpu
from jax.experimental.pallas import tpu_sc as plsc

N = 50000
E = 800000
H = 64
HH = 32
G = 128
NCORE = 2
NSUB = 16
CHUNK = 128

NCH_C = 392
EPAD = NSUB * NCH_C * CHUNK
NCH_A = NCH_C // 2
KA = 14
IB = 56
NB = NCH_C // IB
NBUF = 4
GLA = 2
SUP = IB // NBUF

ACC_ROWS = 50048
DW = 16
ZROWS = ACC_ROWS // NSUB

RB = 400
GRID = N // RB

@functools.cache
def _make_deg_kernel():
    return functools.partial(
        pl.kernel,
        out_type=jax.ShapeDtypeStruct((2, NCORE, ACC_ROWS, DW), jnp.float32),
        mesh=plsc.VectorSubcoreMesh(
            core_axis_name="c", subcore_axis_name="s", num_cores=NCORE,
            num_subcores=NSUB),
        scratch_types=[
            pltpu.VMEM((NCH_A, CHUNK), jnp.int32),
            pltpu.VMEM((CHUNK, DW), jnp.float32),
            pltpu.VMEM_SHARED((ACC_ROWS, DW), jnp.float32),
            pltpu.SemaphoreType.DMA,
        ],
        compiler_params=pltpu.CompilerParams(use_tc_tiling_on_sc=False),
    )(_deg_body)


def _deg_body(dst_hbm, ones_hbm, zeros_hbm, out_hbm, dst_v, ones_v, acc,
              sem):
    c = lax.axis_index("c")
    s = lax.axis_index("s")
    pltpu.sync_copy(ones_hbm, ones_v)
    for t in range(2):
        pltpu.sync_copy(zeros_hbm, acc.at[pl.ds(s * ZROWS, ZROWS)])
        pltpu.sync_copy(dst_hbm.at[t, c, s], dst_v)
        plsc.subcore_barrier()

        def body(m, carry):
            for k in range(KA):
                pltpu.async_copy(ones_v, acc.at[dst_v.at[m * KA + k]], sem,
                                 add=True)
            for k in range(KA):
                pltpu.make_async_copy(ones_v, acc.at[dst_v.at[m * KA + k]],
                                      sem).wait()
            return carry

        lax.fori_loop(0, NCH_A // KA, body, 0)
        plsc.subcore_barrier()
        pltpu.sync_copy(acc.at[pl.ds(s * ZROWS, ZROWS)],
                        out_hbm.at[t, c, pl.ds(s * ZROWS, ZROWS)])
        plsc.subcore_barrier()


@functools.cache
def _make_edge_kernel():
    return functools.partial(
        pl.kernel,
        out_type=jax.ShapeDtypeStruct((NCORE, ACC_ROWS, HH), jnp.float32),
        mesh=plsc.VectorSubcoreMesh(
            core_axis_name="c", subcore_axis_name="s", num_cores=NCORE,
            num_subcores=NSUB),
        scratch_types=[
            pltpu.VMEM((IB, CHUNK), jnp.int32),
            pltpu.VMEM((IB, CHUNK), jnp.int32),
            pltpu.VMEM((NBUF, CHUNK, HH), jnp.float32),
            pltpu.VMEM_SHARED((ACC_ROWS, HH), jnp.float32),
        ] + [pltpu.SemaphoreType.DMA] * (2 * NBUF),
        compiler_params=pltpu.CompilerParams(use_tc_tiling_on_sc=False),
    )(_edge_body)


def _edge_body(src_hbm, dst_hbm, hs_hbm, zeros_hbm, out_hbm,
               src_v, dst_v, rows_v, acc, *sems):
    gsems = sems[:NBUF]
    ssems = sems[NBUF:]
    c = lax.axis_index("c")
    s = lax.axis_index("s")
    pltpu.sync_copy(zeros_hbm, acc.at[pl.ds(s * ZROWS, ZROWS)])
    plsc.subcore_barrier()

    def gather(j, k):
        pltpu.async_copy(hs_hbm.at[src_v.at[j]], rows_v.at[k], gsems[k])

    def gather_wait(j, k):
        pltpu.make_async_copy(hs_hbm.at[src_v.at[j]], rows_v.at[k],
                              gsems[k]).wait()

    def scat(j, k):
        pltpu.async_copy(rows_v.at[k], acc.at[dst_v.at[j]], ssems[k],
                         add=True)

    def scat_wait(j, k):
        pltpu.make_async_copy(rows_v.at[k], acc.at[dst_v.at[j]],
                              ssems[k]).wait()

    def block(b, carry):
        pltpu.sync_copy(src_hbm.at[c, s, pl.ds(b * IB, IB)], src_v)
        pltpu.sync_copy(dst_hbm.at[s, pl.ds(b * IB, IB)], dst_v)
        for k in range(GLA):
            gather(k, k)

        def sup(m, carry2):
            for k in range(NBUF):
                j = m * NBUF + k
                gather_wait(j, k)
                scat(j, k)
                kg = (k + GLA) % NBUF

                @pl.when(j + GLA < IB)
                def _(j=j, kg=kg):
                    @pl.when(j + GLA - NBUF >= 0)
                    def _():
                        scat_wait(j + GLA - NBUF, kg)

                    gather(j + GLA, kg)
            return carry2

        lax.fori_loop(0, SUP, sup, 0)
        for k in range(NBUF):
            scat_wait(IB - NBUF + k, k)
        return carry

    lax.fori_loop(0, NB, block, 0)
    plsc.subcore_barrier()
    pltpu.sync_copy(acc.at[pl.ds(s * ZROWS, ZROWS)],
                    out_hbm.at[c, pl.ds(s * ZROWS, ZROWS)])


def _prescale_body(emb_ref, degp_ref, w1_ref, w2_ref, hs_ref):
    emb = emb_ref[...]
    for t, w_ref in enumerate((w1_ref, w2_ref)):
        deg = degp_ref[t, 0][:, :1] + degp_ref[t, 1][:, :1] + 1.0
        dinv = lax.rsqrt(deg)
        hs = jnp.dot(emb, w_ref[...],
                     preferred_element_type=jnp.float32) * dinv
        hs_ref[t, 0] = hs[:, :HH]
        hs_ref[t, 1] = hs[:, HH:]


def _run_prescale(emb, degp, w1, w2):
    return pl.pallas_call(
        _prescale_body,
        grid=(GRID,),
        in_specs=[
            pl.BlockSpec((RB, H), lambda i: (i, 0)),
            pl.BlockSpec((2, NCORE, RB, DW), lambda i: (0, 0, i, 0)),
            pl.BlockSpec((H, H), lambda i: (0, 0)),
            pl.BlockSpec((H, H), lambda i: (0, 0)),
        ],
        out_specs=pl.BlockSpec((2, NCORE, RB, HH), lambda i: (0, 0, i, 0)),
        out_shape=jax.ShapeDtypeStruct((2, NCORE, N, HH), jnp.float32),
    )(emb, degp, w1, w2)


def _final_body(a10_ref, a11_ref, a20_ref, a21_ref, hs1_ref, hs2_ref,
                degp_ref, batch_ref, b_ref, fcw_ref, fcb_ref, out_ref,
                acc_ref, cnt_ref):
    i = pl.program_id(0)

    @pl.when(i == 0)
    def _():
        acc_ref[...] = jnp.zeros_like(acc_ref)
        cnt_ref[...] = jnp.zeros_like(cnt_ref)

    h = b_ref[...]
    for t, (aggs, hs_ref) in enumerate((((a10_ref, a11_ref), hs1_ref),
                                        ((a20_ref, a21_ref), hs2_ref))):
        deg = degp_ref[t, 0][:, :1] + degp_ref[t, 1][:, :1] + 1.0
        dinv = lax.rsqrt(deg)
        agg = jnp.concatenate([aggs[0][0], aggs[1][0]], axis=1)
        hs = jnp.concatenate([hs_ref[0, 0], hs_ref[0, 1]], axis=1)
        h = h + (agg + hs) * dinv

    seg = batch_ref[0]
    onehot = (lax.broadcasted_iota(jnp.int32, (G, RB), 0) ==
              seg).astype(jnp.float32)
    acc_ref[...] += jnp.dot(onehot, h, preferred_element_type=jnp.float32)
    cnt_ref[...] += jnp.sum(onehot, axis=1, keepdims=True)

    @pl.when(i == GRID - 1)
    def _():
        pooled = acc_ref[...] / jnp.maximum(cnt_ref[...], 1.0)
        logits = jnp.dot(pooled, fcw_ref[...],
                         preferred_element_type=jnp.float32) + fcb_ref[...]
        out_ref[...] = 1.0 / (1.0 + jnp.exp(-logits))


def _run_final(agg1, agg2, hs, degp, batch3, b, fcw, fcb):
    return pl.pallas_call(
        _final_body,
        grid=(GRID,),
        in_specs=[
            pl.BlockSpec((1, RB, HH), lambda i: (0, i, 0)),
            pl.BlockSpec((1, RB, HH), lambda i: (1, i, 0)),
            pl.BlockSpec((1, RB, HH), lambda i: (0, i, 0)),
            pl.BlockSpec((1, RB, HH), lambda i: (1, i, 0)),
            pl.BlockSpec((1, NCORE, RB, HH), lambda i: (0, 0, i, 0)),
            pl.BlockSpec((1, NCORE, RB, HH), lambda i: (1, 0, i, 0)),
            pl.BlockSpec((2, NCORE, RB, DW), lambda i: (0, 0, i, 0)),
            pl.BlockSpec((1, 1, RB), lambda i: (i, 0, 0)),
            pl.BlockSpec((1, H), lambda i: (0, 0)),
            pl.BlockSpec((H, 1), lambda i: (0, 0)),
            pl.BlockSpec((1, 1), lambda i: (0, 0)),
        ],
        out_specs=pl.BlockSpec((G, 1), lambda i: (0, 0)),
        out_shape=jax.ShapeDtypeStruct((G, 1), jnp.float32),
        scratch_shapes=[
            pltpu.VMEM((G, H), jnp.float32),
            pltpu.VMEM((G, 1), jnp.float32),
        ],
    )(agg1, agg1, agg2, agg2, hs, hs, degp, batch3, b, fcw, fcb)


def kernel(x, edge_index_t1, edge_index_t2, batch, emb_table,
           W1, b1, W2, b2, fc_W, fc_b):
    pad = EPAD - E
    pad_src = jnp.zeros((pad,), jnp.int32)
    pad_dst = jnp.full((pad,), N, jnp.int32)

    def prep(ei, t):
        srcr = jnp.concatenate([ei[0], pad_src]).reshape(NSUB, NCH_C, CHUNK)
        dst = jnp.concatenate([ei[1], pad_dst])
        src_adj = jnp.stack([srcr + (2 * t + cc) * N for cc in range(NCORE)])
        dst_c = dst.reshape(NSUB, NCH_C, CHUNK)
        dst_a = dst.reshape(NCORE, NSUB, NCH_A, CHUNK)
        return src_adj, dst_c, dst_a

    src1, dstc1, dsta1 = prep(edge_index_t1, 0)
    src2, dstc2, dsta2 = prep(edge_index_t2, 1)
    dsta_all = jnp.stack([dsta1, dsta2])

    ones_a = jnp.ones((CHUNK, DW), jnp.float32)
    zeros_a = jnp.zeros((ZROWS, DW), jnp.float32)
    zeros_c = jnp.zeros((ZROWS, HH), jnp.float32)

    degp = _make_deg_kernel()(dsta_all, ones_a, zeros_a)
    hs = _run_prescale(emb_table, degp, W1, W2)
    hs_cat = hs.reshape(2 * NCORE * N, HH)

    edge_kernel = _make_edge_kernel()
    agg1 = edge_kernel(src1, dstc1, hs_cat, zeros_c)
    agg2 = edge_kernel(src2, dstc2, hs_cat, zeros_c)

    batch3 = batch.reshape(GRID, 1, RB)
    b = (b1 + b2).reshape(1, H)
    return _run_final(agg1, agg2, hs, degp, batch3, b,
                      fc_W, fc_b.reshape(1, 1))


# speedup vs baseline: 1.0634x; 1.0634x over previous
import functools

import jax
import jax.numpy as jnp
from jax import lax
from jax.experimental import pallas as pl
from jax.experimental.pallas import tpu as pltpu
from jax.experimental.pallas import tpu_sc as plsc

N = 50000
E = 800000
H = 64
HH = 32
G = 128
NCORE = 2
NSUB = 16
CHUNK = 128

NCH_C = 392
EPAD = NSUB * NCH_C * CHUNK
NCH_A = NCH_C // 2
KA = 14
IB = 56
NB = NCH_C // IB
NBUF = 4
SUP = IB // NBUF

ACC_ROWS = 50048
DW = 16
ZROWS = ACC_ROWS // NSUB

RB = 400
GRID = N // RB

@functools.cache
def _make_deg_kernel():
    return functools.partial(
        pl.kernel,
        out_type=jax.ShapeDtypeStruct((2, NCORE, ACC_ROWS, DW), jnp.float32),
        mesh=plsc.VectorSubcoreMesh(
            core_axis_name="c", subcore_axis_name="s", num_cores=NCORE,
            num_subcores=NSUB),
        scratch_types=[
            pltpu.VMEM((NCH_A, CHUNK), jnp.int32),
            pltpu.VMEM((CHUNK, DW), jnp.float32),
            pltpu.VMEM_SHARED((ACC_ROWS, DW), jnp.float32),
            pltpu.SemaphoreType.DMA,
        ],
        compiler_params=pltpu.CompilerParams(use_tc_tiling_on_sc=False),
    )(_deg_body)


def _deg_body(dst_hbm, ones_hbm, zeros_hbm, out_hbm, dst_v, ones_v, acc,
              sem):
    c = lax.axis_index("c")
    s = lax.axis_index("s")
    pltpu.sync_copy(ones_hbm, ones_v)
    for t in range(2):
        pltpu.sync_copy(zeros_hbm, acc.at[pl.ds(s * ZROWS, ZROWS)])
        pltpu.sync_copy(dst_hbm.at[t, c, s], dst_v)
        plsc.subcore_barrier()

        def body(m, carry):
            for k in range(KA):
                pltpu.async_copy(ones_v, acc.at[dst_v.at[m * KA + k]], sem,
                                 add=True)
            for k in range(KA):
                pltpu.make_async_copy(ones_v, acc.at[dst_v.at[m * KA + k]],
                                      sem).wait()
            return carry

        lax.fori_loop(0, NCH_A // KA, body, 0)
        plsc.subcore_barrier()
        pltpu.sync_copy(acc.at[pl.ds(s * ZROWS, ZROWS)],
                        out_hbm.at[t, c, pl.ds(s * ZROWS, ZROWS)])
        plsc.subcore_barrier()


@functools.cache
def _make_edge_kernel():
    return functools.partial(
        pl.kernel,
        out_type=jax.ShapeDtypeStruct((2, NCORE, ACC_ROWS, HH), jnp.float32),
        mesh=plsc.VectorSubcoreMesh(
            core_axis_name="c", subcore_axis_name="s", num_cores=NCORE,
            num_subcores=NSUB),
        scratch_types=[
            pltpu.VMEM((IB, CHUNK), jnp.int32),
            pltpu.VMEM((IB, CHUNK), jnp.int32),
            pltpu.VMEM((NBUF, CHUNK, HH), jnp.float32),
            pltpu.VMEM_SHARED((ACC_ROWS, HH), jnp.float32),
        ] + [pltpu.SemaphoreType.DMA] * NBUF,
        compiler_params=pltpu.CompilerParams(use_tc_tiling_on_sc=False),
    )(_edge_body)


def _edge_body(src_hbm, dst_hbm, hs_hbm, zeros_hbm, out_hbm,
               src_v, dst_v, rows_v, acc, *sems):
    c = lax.axis_index("c")
    s = lax.axis_index("s")

    def phase(t, carry0):
        pltpu.sync_copy(zeros_hbm, acc.at[pl.ds(s * ZROWS, ZROWS)])
        plsc.subcore_barrier()

        def block(b, carry):
            pltpu.sync_copy(src_hbm.at[t, c, s, pl.ds(b * IB, IB)], src_v)
            pltpu.sync_copy(dst_hbm.at[t, s, pl.ds(b * IB, IB)], dst_v)
            for k in range(NBUF):
                pltpu.async_copy(hs_hbm.at[src_v.at[k]], rows_v.at[k],
                                 sems[k])

            def sup(m, carry2):
                for k in range(NBUF):
                    j = m * NBUF + k
                    pltpu.make_async_copy(hs_hbm.at[src_v.at[j]],
                                          rows_v.at[k], sems[k]).wait()
                    pltpu.sync_copy(rows_v.at[k], acc.at[dst_v.at[j]],
                                    add=True)

                    @pl.when(j + NBUF < IB)
                    def _(k=k, j=j):
                        pltpu.async_copy(hs_hbm.at[src_v.at[j + NBUF]],
                                         rows_v.at[k], sems[k])
                return carry2

            lax.fori_loop(0, SUP, sup, 0)
            return carry

        lax.fori_loop(0, NB, block, 0)
        plsc.subcore_barrier()
        pltpu.sync_copy(acc.at[pl.ds(s * ZROWS, ZROWS)],
                        out_hbm.at[t, c, pl.ds(s * ZROWS, ZROWS)])
        plsc.subcore_barrier()
        return carry0

    lax.fori_loop(0, 2, phase, 0)


def _prescale_body(emb_ref, degp_ref, w1_ref, w2_ref, hs_ref):
    emb = emb_ref[...]
    for t, w_ref in enumerate((w1_ref, w2_ref)):
        deg = degp_ref[t, 0][:, :1] + degp_ref[t, 1][:, :1] + 1.0
        dinv = lax.rsqrt(deg)
        hs = jnp.dot(emb, w_ref[...],
                     preferred_element_type=jnp.float32) * dinv
        hs_ref[t, 0] = hs[:, :HH]
        hs_ref[t, 1] = hs[:, HH:]


def _run_prescale(emb, degp, w1, w2):
    return pl.pallas_call(
        _prescale_body,
        grid=(GRID,),
        in_specs=[
            pl.BlockSpec((RB, H), lambda i: (i, 0)),
            pl.BlockSpec((2, NCORE, RB, DW), lambda i: (0, 0, i, 0)),
            pl.BlockSpec((H, H), lambda i: (0, 0)),
            pl.BlockSpec((H, H), lambda i: (0, 0)),
        ],
        out_specs=pl.BlockSpec((2, NCORE, RB, HH), lambda i: (0, 0, i, 0)),
        out_shape=jax.ShapeDtypeStruct((2, NCORE, N, HH), jnp.float32),
    )(emb, degp, w1, w2)


def _final_body(a10_ref, a11_ref, a20_ref, a21_ref, hs1_ref, hs2_ref,
                degp_ref, batch_ref, b_ref, fcw_ref, fcb_ref, out_ref,
                acc_ref, cnt_ref):
    i = pl.program_id(0)

    @pl.when(i == 0)
    def _():
        acc_ref[...] = jnp.zeros_like(acc_ref)
        cnt_ref[...] = jnp.zeros_like(cnt_ref)

    h = b_ref[...]
    for t, (aggs, hs_ref) in enumerate((((a10_ref, a11_ref), hs1_ref),
                                        ((a20_ref, a21_ref), hs2_ref))):
        deg = degp_ref[t, 0][:, :1] + degp_ref[t, 1][:, :1] + 1.0
        dinv = lax.rsqrt(deg)
        agg = jnp.concatenate([aggs[0][0, 0], aggs[1][0, 0]], axis=1)
        hs = jnp.concatenate([hs_ref[0, 0], hs_ref[0, 1]], axis=1)
        h = h + (agg + hs) * dinv

    seg = batch_ref[0]
    onehot = (lax.broadcasted_iota(jnp.int32, (G, RB), 0) ==
              seg).astype(jnp.float32)
    acc_ref[...] += jnp.dot(onehot, h, preferred_element_type=jnp.float32)
    cnt_ref[...] += jnp.sum(onehot, axis=1, keepdims=True)

    @pl.when(i == GRID - 1)
    def _():
        pooled = acc_ref[...] / jnp.maximum(cnt_ref[...], 1.0)
        logits = jnp.dot(pooled, fcw_ref[...],
                         preferred_element_type=jnp.float32) + fcb_ref[...]
        out_ref[...] = 1.0 / (1.0 + jnp.exp(-logits))


def _run_final(agg, hs, degp, batch3, b, fcw, fcb):
    return pl.pallas_call(
        _final_body,
        grid=(GRID,),
        in_specs=[
            pl.BlockSpec((1, 1, RB, HH), lambda i: (0, 0, i, 0)),
            pl.BlockSpec((1, 1, RB, HH), lambda i: (0, 1, i, 0)),
            pl.BlockSpec((1, 1, RB, HH), lambda i: (1, 0, i, 0)),
            pl.BlockSpec((1, 1, RB, HH), lambda i: (1, 1, i, 0)),
            pl.BlockSpec((1, NCORE, RB, HH), lambda i: (0, 0, i, 0)),
            pl.BlockSpec((1, NCORE, RB, HH), lambda i: (1, 0, i, 0)),
            pl.BlockSpec((2, NCORE, RB, DW), lambda i: (0, 0, i, 0)),
            pl.BlockSpec((1, 1, RB), lambda i: (i, 0, 0)),
            pl.BlockSpec((1, H), lambda i: (0, 0)),
            pl.BlockSpec((H, 1), lambda i: (0, 0)),
            pl.BlockSpec((1, 1), lambda i: (0, 0)),
        ],
        out_specs=pl.BlockSpec((G, 1), lambda i: (0, 0)),
        out_shape=jax.ShapeDtypeStruct((G, 1), jnp.float32),
        scratch_shapes=[
            pltpu.VMEM((G, H), jnp.float32),
            pltpu.VMEM((G, 1), jnp.float32),
        ],
    )(agg, agg, agg, agg, hs, hs, degp, batch3, b, fcw, fcb)


def kernel(x, edge_index_t1, edge_index_t2, batch, emb_table,
           W1, b1, W2, b2, fc_W, fc_b):
    pad = EPAD - E
    pad_src = jnp.zeros((pad,), jnp.int32)
    pad_dst = jnp.full((pad,), N, jnp.int32)

    def prep(ei, t):
        srcr = jnp.concatenate([ei[0], pad_src]).reshape(NSUB, NCH_C, CHUNK)
        dst = jnp.concatenate([ei[1], pad_dst])
        src_adj = jnp.stack([srcr + (2 * t + cc) * N for cc in range(NCORE)])
        dst_c = dst.reshape(NSUB, NCH_C, CHUNK)
        dst_a = dst.reshape(NCORE, NSUB, NCH_A, CHUNK)
        return src_adj, dst_c, dst_a

    src1, dstc1, dsta1 = prep(edge_index_t1, 0)
    src2, dstc2, dsta2 = prep(edge_index_t2, 1)
    dsta_all = jnp.stack([dsta1, dsta2])

    ones_a = jnp.ones((CHUNK, DW), jnp.float32)
    zeros_a = jnp.zeros((ZROWS, DW), jnp.float32)
    zeros_c = jnp.zeros((ZROWS, HH), jnp.float32)

    degp = _make_deg_kernel()(dsta_all, ones_a, zeros_a)
    hs = _run_prescale(emb_table, degp, W1, W2)
    hs_cat = hs.reshape(2 * NCORE * N, HH)

    src_all = jnp.stack([src1, src2])
    dstc_all = jnp.stack([dstc1, dstc2])
    agg = _make_edge_kernel()(src_all, dstc_all, hs_cat, zeros_c)

    batch3 = batch.reshape(GRID, 1, RB)
    b = (b1 + b2).reshape(1, H)
    return _run_final(agg, hs, degp, batch3, b,
                      fc_W, fc_b.reshape(1, 1))


# speedup vs baseline: 1.0945x; 1.0292x over previous
import functools

import jax
import jax.numpy as jnp
from jax import lax
from jax.experimental import pallas as pl
from jax.experimental.pallas import tpu as pltpu
from jax.experimental.pallas import tpu_sc as plsc

N = 50000
E = 800000
H = 64
HH = 32
G = 128
NCORE = 2
NSUB = 16
CHUNK = 128

NCH_C = 392
EPAD = NSUB * NCH_C * CHUNK
NCH_A = NCH_C // 2
KA = 14
IB = 56
NB = NCH_C // IB
NBUF = 4
SUP = IB // NBUF

ACC_ROWS = 50048
DW = 16
ZROWS = ACC_ROWS // NSUB

RB = 400
GRID = N // RB

@functools.cache
def _make_deg_kernel():
    return functools.partial(
        pl.kernel,
        out_type=jax.ShapeDtypeStruct((2, NCORE, ACC_ROWS, DW), jnp.float32),
        mesh=plsc.VectorSubcoreMesh(
            core_axis_name="c", subcore_axis_name="s", num_cores=NCORE,
            num_subcores=NSUB),
        scratch_types=[
            pltpu.VMEM((NCH_A, CHUNK), jnp.int32),
            pltpu.VMEM((CHUNK, DW), jnp.float32),
            pltpu.VMEM_SHARED((ACC_ROWS, DW), jnp.float32),
            pltpu.SemaphoreType.DMA,
        ],
        compiler_params=pltpu.CompilerParams(use_tc_tiling_on_sc=False),
    )(_deg_body)


def _deg_body(dst_hbm, ones_hbm, zeros_hbm, out_hbm, dst_v, ones_v, acc,
              sem):
    c = lax.axis_index("c")
    s = lax.axis_index("s")
    pltpu.sync_copy(ones_hbm, ones_v)
    for t in range(2):
        pltpu.sync_copy(zeros_hbm, acc.at[pl.ds(s * ZROWS, ZROWS)])
        pltpu.sync_copy(dst_hbm.at[t, c, s], dst_v)
        plsc.subcore_barrier()

        def body(m, carry):
            for k in range(KA):
                pltpu.async_copy(ones_v, acc.at[dst_v.at[m * KA + k]], sem,
                                 add=True)
            for k in range(KA):
                pltpu.make_async_copy(ones_v, acc.at[dst_v.at[m * KA + k]],
                                      sem).wait()
            return carry

        lax.fori_loop(0, NCH_A // KA, body, 0)
        plsc.subcore_barrier()
        pltpu.sync_copy(acc.at[pl.ds(s * ZROWS, ZROWS)],
                        out_hbm.at[t, c, pl.ds(s * ZROWS, ZROWS)])
        plsc.subcore_barrier()


@functools.cache
def _make_edge_kernel():
    return functools.partial(
        pl.kernel,
        out_type=jax.ShapeDtypeStruct((NCORE, ACC_ROWS, HH), jnp.float32),
        mesh=plsc.VectorSubcoreMesh(
            core_axis_name="c", subcore_axis_name="s", num_cores=NCORE,
            num_subcores=NSUB),
        scratch_types=[
            pltpu.VMEM((IB, CHUNK), jnp.int32),
            pltpu.VMEM((IB, CHUNK), jnp.int32),
            pltpu.VMEM((NBUF, CHUNK, HH), jnp.float32),
            pltpu.VMEM_SHARED((ACC_ROWS, HH), jnp.float32),
        ] + [pltpu.SemaphoreType.DMA] * NBUF,
        compiler_params=pltpu.CompilerParams(use_tc_tiling_on_sc=False),
    )(_edge_body)


def _edge_body(src_hbm, dst_hbm, hs_hbm, zeros_hbm, out_hbm,
               src_v, dst_v, rows_v, acc, *sems):
    c = lax.axis_index("c")
    s = lax.axis_index("s")
    pltpu.sync_copy(zeros_hbm, acc.at[pl.ds(s * ZROWS, ZROWS)])
    plsc.subcore_barrier()

    def block(b, carry):
        pltpu.sync_copy(src_hbm.at[c, s, pl.ds(b * IB, IB)], src_v)
        pltpu.sync_copy(dst_hbm.at[s, pl.ds(b * IB, IB)], dst_v)
        for k in range(NBUF):
            pltpu.async_copy(hs_hbm.at[src_v.at[k]], rows_v.at[k], sems[k])

        def sup(m, carry2):
            for k in range(NBUF):
                j = m * NBUF + k
                pltpu.make_async_copy(hs_hbm.at[src_v.at[j]], rows_v.at[k],
                                      sems[k]).wait()
                pltpu.sync_copy(rows_v.at[k], acc.at[dst_v.at[j]], add=True)

                @pl.when(j + NBUF < IB)
                def _(k=k, j=j):
                    pltpu.async_copy(hs_hbm.at[src_v.at[j + NBUF]],
                                     rows_v.at[k], sems[k])
            return carry2

        lax.fori_loop(0, SUP, sup, 0)
        return carry

    lax.fori_loop(0, NB, block, 0)
    plsc.subcore_barrier()
    pltpu.sync_copy(acc.at[pl.ds(s * ZROWS, ZROWS)],
                    out_hbm.at[c, pl.ds(s * ZROWS, ZROWS)])


def _prescale_body(emb_ref, degp_ref, w1_ref, w2_ref, hs_ref):
    emb = emb_ref[...]
    for t, w_ref in enumerate((w1_ref, w2_ref)):
        deg = degp_ref[t, 0][:, :1] + degp_ref[t, 1][:, :1] + 1.0
        dinv = lax.rsqrt(deg)
        hs = jnp.dot(emb, w_ref[...],
                     preferred_element_type=jnp.float32) * dinv
        hs_ref[t, 0] = hs[:, :HH]
        hs_ref[t, 1] = hs[:, HH:]


def _run_prescale(emb, degp, w1, w2):
    return pl.pallas_call(
        _prescale_body,
        grid=(GRID,),
        in_specs=[
            pl.BlockSpec((RB, H), lambda i: (i, 0)),
            pl.BlockSpec((2, NCORE, RB, DW), lambda i: (0, 0, i, 0)),
            pl.BlockSpec((H, H), lambda i: (0, 0)),
            pl.BlockSpec((H, H), lambda i: (0, 0)),
        ],
        out_specs=pl.BlockSpec((2, NCORE, RB, HH), lambda i: (0, 0, i, 0)),
        out_shape=jax.ShapeDtypeStruct((2, NCORE, N, HH), jnp.float32),
    )(emb, degp, w1, w2)


def _final_body(a10_ref, a11_ref, a20_ref, a21_ref, hs1_ref, hs2_ref,
                degp_ref, batch_ref, b_ref, fcw_ref, fcb_ref, out_ref,
                acc_ref, cnt_ref):
    i = pl.program_id(0)

    @pl.when(i == 0)
    def _():
        acc_ref[...] = jnp.zeros_like(acc_ref)
        cnt_ref[...] = jnp.zeros_like(cnt_ref)

    h = b_ref[...]
    for t, (aggs, hs_ref) in enumerate((((a10_ref, a11_ref), hs1_ref),
                                        ((a20_ref, a21_ref), hs2_ref))):
        deg = degp_ref[t, 0][:, :1] + degp_ref[t, 1][:, :1] + 1.0
        dinv = lax.rsqrt(deg)
        agg = jnp.concatenate([aggs[0][0], aggs[1][0]], axis=1)
        hs = jnp.concatenate([hs_ref[0, 0], hs_ref[0, 1]], axis=1)
        h = h + (agg + hs) * dinv

    seg = batch_ref[0]
    onehot = (lax.broadcasted_iota(jnp.int32, (G, RB), 0) ==
              seg).astype(jnp.float32)
    acc_ref[...] += jnp.dot(onehot, h, preferred_element_type=jnp.float32)
    cnt_ref[...] += jnp.sum(onehot, axis=1, keepdims=True)

    @pl.when(i == GRID - 1)
    def _():
        pooled = acc_ref[...] / jnp.maximum(cnt_ref[...], 1.0)
        logits = jnp.dot(pooled, fcw_ref[...],
                         preferred_element_type=jnp.float32) + fcb_ref[...]
        out_ref[...] = 1.0 / (1.0 + jnp.exp(-logits))


def _run_final(agg1, agg2, hs, degp, batch3, b, fcw, fcb):
    return pl.pallas_call(
        _final_body,
        grid=(GRID,),
        in_specs=[
            pl.BlockSpec((1, RB, HH), lambda i: (0, i, 0)),
            pl.BlockSpec((1, RB, HH), lambda i: (1, i, 0)),
            pl.BlockSpec((1, RB, HH), lambda i: (0, i, 0)),
            pl.BlockSpec((1, RB, HH), lambda i: (1, i, 0)),
            pl.BlockSpec((1, NCORE, RB, HH), lambda i: (0, 0, i, 0)),
            pl.BlockSpec((1, NCORE, RB, HH), lambda i: (1, 0, i, 0)),
            pl.BlockSpec((2, NCORE, RB, DW), lambda i: (0, 0, i, 0)),
            pl.BlockSpec((1, 1, RB), lambda i: (i, 0, 0)),
            pl.BlockSpec((1, H), lambda i: (0, 0)),
            pl.BlockSpec((H, 1), lambda i: (0, 0)),
            pl.BlockSpec((1, 1), lambda i: (0, 0)),
        ],
        out_specs=pl.BlockSpec((G, 1), lambda i: (0, 0)),
        out_shape=jax.ShapeDtypeStruct((G, 1), jnp.float32),
        scratch_shapes=[
            pltpu.VMEM((G, H), jnp.float32),
            pltpu.VMEM((G, 1), jnp.float32),
        ],
    )(agg1, agg1, agg2, agg2, hs, hs, degp, batch3, b, fcw, fcb)


def kernel(x, edge_index_t1, edge_index_t2, batch, emb_table,
           W1, b1, W2, b2, fc_W, fc_b):
    pad = EPAD - E
    pad_src = jnp.zeros((pad,), jnp.int32)
    pad_dst = jnp.full((pad,), N, jnp.int32)

    def prep(ei, t):
        srcr = jnp.concatenate([ei[0], pad_src]).reshape(NSUB, NCH_C, CHUNK)
        dst = jnp.concatenate([ei[1], pad_dst])
        src_adj = jnp.stack([srcr + (2 * t + cc) * N for cc in range(NCORE)])
        dst_c = dst.reshape(NSUB, NCH_C, CHUNK)
        dst_a = dst.reshape(NCORE, NSUB, NCH_A, CHUNK)
        return src_adj, dst_c, dst_a

    src1, dstc1, dsta1 = prep(edge_index_t1, 0)
    src2, dstc2, dsta2 = prep(edge_index_t2, 1)
    dsta_all = jnp.stack([dsta1, dsta2])

    ones_a = jnp.ones((CHUNK, DW), jnp.float32)
    zeros_a = jnp.zeros((ZROWS, DW), jnp.float32)
    zeros_c = jnp.zeros((ZROWS, HH), jnp.float32)

    degp = _make_deg_kernel()(dsta_all, ones_a, zeros_a)
    hs = _run_prescale(emb_table, degp, W1, W2)
    hs_cat = hs.reshape(2 * NCORE * N, HH)

    edge_kernel = _make_edge_kernel()
    agg1 = edge_kernel(src1, dstc1, hs_cat, zeros_c)
    agg2 = edge_kernel(src2, dstc2, hs_cat, zeros_c)

    batch3 = batch.reshape(GRID, 1, RB)
    b = (b1 + b2).reshape(1, H)
    return _run_final(agg1, agg2, hs, degp, batch3, b,
                      fc_W, fc_b.reshape(1, 1))
